# baseline (speedup 1.0000x reference)
import functools

import jax
import jax.numpy as jnp
from jax import lax
from jax.experimental import pallas as pl
from jax.experimental.pallas import tpu as pltpu
from jax.experimental.pallas import tpu_sc as plsc

NUM_E = 100000
D = 300
DP = 304
BATCH = 4096
HIST = 50
B = BATCH * HIST
NC, NS = 2, 16
NW = NC * NS
CHUNK = 128
CPW = B // NW // CHUNK

_mesh = plsc.VectorSubcoreMesh(core_axis_name="c", subcore_axis_name="s")


@functools.partial(
    pl.kernel,
    mesh=_mesh,
    out_type=jax.ShapeDtypeStruct((B, DP), jnp.float32),
    scratch_types=[
        pltpu.VMEM((CPW, CHUNK), jnp.int32),
        pltpu.VMEM((CHUNK, DP), jnp.float32),
        pltpu.SemaphoreType.DMA,
    ],
    compiler_params=pltpu.CompilerParams(use_tc_tiling_on_sc=False),
)
def _gather(x_hbm, w_hbm, out_hbm, idx_v, rows_v, sem):
    wid = lax.axis_index("s") * NC + lax.axis_index("c")
    pltpu.sync_copy(x_hbm.at[wid], idx_v)
    base = wid * (CPW * CHUNK)

    def body(j, carry):
        pltpu.async_copy(w_hbm.at[idx_v.at[j]], rows_v, sem).wait()
        pltpu.sync_copy(rows_v, out_hbm.at[pl.ds(base + j * CHUNK, CHUNK)])
        return carry

    lax.fori_loop(0, CPW, body, 0)


def kernel(x, weight):
    xr = x.reshape(NW, CPW, CHUNK)
    wp = jnp.pad(weight, ((0, 0), (0, DP - D)))
    out = _gather(xr, wp)
    return out[:, :D].reshape(BATCH, HIST, D)


# speedup vs baseline: 1.3571x; 1.3571x over previous
import functools

import jax
import jax.numpy as jnp
from jax import lax
from jax.experimental import pallas as pl
from jax.experimental.pallas import tpu as pltpu
from jax.experimental.pallas import tpu_sc as plsc

NUM_E = 100000
D = 300
DP = 384
BATCH = 4096
HIST = 50
B = BATCH * HIST
NC, NS = 2, 16
NW = NC * NS
CHUNK = 128
CPW = B // NW // CHUNK

_mesh = plsc.VectorSubcoreMesh(core_axis_name="c", subcore_axis_name="s")


@functools.partial(
    pl.kernel,
    mesh=_mesh,
    out_type=jax.ShapeDtypeStruct((B, DP), jnp.float32),
    scratch_types=[
        pltpu.VMEM((CPW, CHUNK), jnp.int32),
        pltpu.VMEM((CHUNK, DP), jnp.float32),
        pltpu.SemaphoreType.DMA,
    ],
    compiler_params=pltpu.CompilerParams(use_tc_tiling_on_sc=True),
)
def _gather(x_hbm, w_hbm, out_hbm, idx_v, rows_v, sem):
    wid = lax.axis_index("s") * NC + lax.axis_index("c")
    pltpu.sync_copy(x_hbm.at[wid], idx_v)
    base = wid * (CPW * CHUNK)

    def body(j, carry):
        pltpu.async_copy(w_hbm.at[idx_v.at[j]], rows_v, sem).wait()
        pltpu.sync_copy(rows_v, out_hbm.at[pl.ds(base + j * CHUNK, CHUNK)])
        return carry

    lax.fori_loop(0, CPW, body, 0)


def kernel(x, weight):
    xr = x.reshape(NW, CPW, CHUNK)
    wp = jnp.pad(weight, ((0, 0), (0, DP - D)))
    out = _gather(xr, wp)
    return out[:, :D].reshape(BATCH, HIST, D)


# speedup vs baseline: 1.3577x; 1.0004x over previous
import functools

import jax
import jax.numpy as jnp
from jax import lax
from jax.experimental import pallas as pl
from jax.experimental.pallas import tpu as pltpu
from jax.experimental.pallas import tpu_sc as plsc

NUM_E = 100000
D = 300
DP = 384
BATCH = 4096
HIST = 50
B = BATCH * HIST
NC, NS = 2, 16
NW = NC * NS
CHUNK = 128
CPW = B // NW // CHUNK

_mesh = plsc.VectorSubcoreMesh(core_axis_name="c", subcore_axis_name="s")


@functools.partial(
    pl.kernel,
    mesh=_mesh,
    out_type=jax.ShapeDtypeStruct((B, DP), jnp.float32),
    scratch_types=[
        pltpu.VMEM((CPW, CHUNK), jnp.int32),
        pltpu.VMEM((CHUNK, DP), jnp.float32),
        pltpu.SemaphoreType.DMA,
    ],
    compiler_params=pltpu.CompilerParams(use_tc_tiling_on_sc=True),
)
def _gather(x_hbm, w_hbm, out_hbm, idx_v, rows_v, sem):
    wid = lax.axis_index("s") * NC + lax.axis_index("c")
    pltpu.sync_copy(x_hbm.at[wid], idx_v)
    base = wid * (CPW * CHUNK)

    def body(j, carry):
        pltpu.async_copy(w_hbm.at[idx_v.at[j]], rows_v, sem).wait()
        pltpu.sync_copy(rows_v, out_hbm.at[pl.ds(base + j * CHUNK, CHUNK)])
        return carry

    lax.fori_loop(0, CPW, body, 0)


def kernel(x, weight):
    xr = x.reshape(NW, CPW, CHUNK)
    wp = jnp.pad(weight, ((0, 0), (0, DP - D)))
    out = _gather(xr, wp)
    return out[:, :D].reshape(BATCH, HIST, D) + jnp.float32(0.0)


# speedup vs baseline: 2.1340x; 1.5717x over previous
import functools

import jax
import jax.numpy as jnp
from jax import lax
from jax.experimental import pallas as pl
from jax.experimental.pallas import tpu as pltpu
from jax.experimental.pallas import tpu_sc as plsc

NUM_E = 100000
D = 300
DP = 384
BATCH = 4096
HIST = 50
B = BATCH * HIST
NC, NS = 2, 16
NW = NC * NS
CHUNK = 128
CPW = B // NW // CHUNK

_mesh = plsc.VectorSubcoreMesh(core_axis_name="c", subcore_axis_name="s")


@functools.partial(
    pl.kernel,
    mesh=_mesh,
    out_type=jax.ShapeDtypeStruct((B, DP), jnp.float32),
    scratch_types=[
        pltpu.VMEM((CPW, CHUNK), jnp.int32),
        pltpu.VMEM((CHUNK, DP), jnp.float32),
        pltpu.SemaphoreType.DMA,
    ],
    compiler_params=pltpu.CompilerParams(use_tc_tiling_on_sc=True),
)
def _gather(x_hbm, w_hbm, out_hbm, idx_v, rows_v, sem):
    wid = lax.axis_index("s") * NC + lax.axis_index("c")
    pltpu.sync_copy(x_hbm.at[wid], idx_v)
    base = wid * (CPW * CHUNK)

    def body(j, carry):
        pltpu.async_copy(w_hbm.at[idx_v.at[j]], rows_v, sem).wait()
        pltpu.sync_copy(rows_v, out_hbm.at[pl.ds(base + j * CHUNK, CHUNK)])
        return carry

    lax.fori_loop(0, CPW, body, 0)


_TR_BLOCK = 2048


def _transpose_block(wt_ref, wp_ref):
    wp_ref[...] = jnp.transpose(wt_ref[...], (1, 0))


_transpose = pl.pallas_call(
    _transpose_block,
    grid=(pl.cdiv(NUM_E, _TR_BLOCK),),
    in_specs=[pl.BlockSpec((DP, _TR_BLOCK), lambda i: (0, i))],
    out_specs=pl.BlockSpec((_TR_BLOCK, DP), lambda i: (i, 0)),
    out_shape=jax.ShapeDtypeStruct((NUM_E, DP), jnp.float32),
)


def kernel(x, weight):
    xr = x.reshape(NW, CPW, CHUNK)
    wp = _transpose(lax.transpose(weight, (1, 0)))
    out = _gather(xr, wp)
    return out[:, :D].reshape(BATCH, HIST, D)


# speedup vs baseline: 2.8129x; 1.3182x over previous
import functools

import jax
import jax.numpy as jnp
from jax import lax
from jax.experimental import pallas as pl
from jax.experimental.pallas import tpu as pltpu
from jax.experimental.pallas import tpu_sc as plsc

NUM_E = 100000
D = 300
DP = 384
BATCH = 4096
HIST = 50
B = BATCH * HIST
NC, NS = 2, 16
NW = NC * NS
CHUNK = HIST
BPW = BATCH // NW
CPW = BPW

_mesh = plsc.VectorSubcoreMesh(core_axis_name="c", subcore_axis_name="s")


@functools.partial(
    pl.kernel,
    mesh=_mesh,
    out_type=jax.ShapeDtypeStruct((BATCH, HIST, DP), jnp.float32),
    scratch_types=[
        pltpu.VMEM((CPW, CHUNK), jnp.int32),
        pltpu.VMEM((CHUNK, DP), jnp.float32),
        pltpu.SemaphoreType.DMA,
    ],
    compiler_params=pltpu.CompilerParams(use_tc_tiling_on_sc=True),
)
def _gather(x_hbm, w_hbm, out_hbm, idx_v, rows_v, sem):
    wid = lax.axis_index("s") * NC + lax.axis_index("c")
    pltpu.sync_copy(x_hbm.at[wid], idx_v)
    base = wid * BPW

    def body(j, carry):
        pltpu.async_copy(w_hbm.at[idx_v.at[j]], rows_v, sem).wait()
        pltpu.sync_copy(rows_v, out_hbm.at[base + j])
        return carry

    lax.fori_loop(0, CPW, body, 0)


_TR_BLOCK = 2048


def _transpose_block(wt_ref, wp_ref):
    wp_ref[...] = jnp.transpose(wt_ref[...], (1, 0))


_transpose = pl.pallas_call(
    _transpose_block,
    grid=(pl.cdiv(NUM_E, _TR_BLOCK),),
    in_specs=[pl.BlockSpec((DP, _TR_BLOCK), lambda i: (0, i))],
    out_specs=pl.BlockSpec((_TR_BLOCK, DP), lambda i: (i, 0)),
    out_shape=jax.ShapeDtypeStruct((NUM_E, DP), jnp.float32),
)


def kernel(x, weight):
    xr = x.reshape(NW, CPW, CHUNK)
    wp = _transpose(lax.transpose(weight, (1, 0)))
    out = _gather(xr, wp)
    return out[:, :, :D]
